# baseline (speedup 1.0000x reference)
import functools

import jax
import jax.numpy as jnp
from jax import lax
from jax.experimental import pallas as pl
from jax.experimental.pallas import tpu as pltpu
from jax.experimental.pallas import tpu_sc as plsc

_D = 64
_NW = 32
_CHUNK = 128
_NB = 4
_SCALE = 8.0
_PAIR = 500224


def _prep_body(a_ref, b_ref, o_ref):
    o_ref[...] = jnp.concatenate([a_ref[...].T, b_ref[...].T], axis=1) * _SCALE


def _prep(wt):
    n_blocks = _PAIR // 128
    return pl.pallas_call(
        _prep_body,
        grid=(n_blocks,),
        in_specs=[
            pl.BlockSpec((_D, 128), lambda n: (0, n)),
            pl.BlockSpec((_D, 128), lambda n: (0, n + _PAIR // 128)),
        ],
        out_specs=pl.BlockSpec((128, 128), lambda n: (n, 0)),
        out_shape=jax.ShapeDtypeStruct((_PAIR, 128), jnp.float32),
    )(wt, wt)


def _emb_body(idx_hbm, table_hbm, out_hbm, idx_v, bufg, bufo, *sems):
    n_chunks = idx_v.shape[0]
    n_groups = n_chunks // _NB
    sem_g, sem_o = sems[:_NB], sems[_NB:]
    wid = lax.axis_index("s") * 2 + lax.axis_index("c")
    pltpu.sync_copy(idx_hbm.at[wid], idx_v)
    lane = lax.iota(jnp.int32, 16)

    def fix_idx(j, c):
        for g in range(_CHUNK // 16):
            s = pl.ds(g * 16, 16)
            v = idx_v[j, s]
            idx_v[j, s] = jnp.where(v < _PAIR, v + v, v + v - (2 * _PAIR - 1))
        return c

    lax.fori_loop(0, n_chunks, fix_idx, 0)

    def gather_start(j, b):
        pltpu.async_copy(table_hbm.at[idx_v.at[j]], bufg.at[b], sem_g[b])

    def gather_wait(j, b):
        pltpu.make_async_copy(table_hbm.at[idx_v.at[j]], bufg.at[b],
                              sem_g[b]).wait()

    def out_start(j, b):
        for a in range(8):
            pltpu.async_copy(bufo.at[b].at[a],
                             out_hbm.at[j * 256 + a * 32 + wid], sem_o[b])

    def out_wait(j, b):
        for a in range(8):
            pltpu.make_async_copy(bufo.at[b].at[a],
                                  out_hbm.at[j * 256 + a * 32 + wid],
                                  sem_o[b]).wait()

    def transpose_chunk(b):
        src, dst = bufg.at[b], bufo.at[b]

        def tile(q, c):
            c0 = (q & 7) * 16
            col0 = (q >> 3) * 16
            rows = c0 + lane
            for d in range(16):
                k = col0 + ((lane + d) & 15)
                vals = plsc.load_gather(src, [rows, k])
                plsc.store_scatter(dst, [k >> 3, ((k & 7) << 7) + rows], vals)
            return c

        lax.fori_loop(0, 32, tile, 0)

    def group(g, first, fire):
        for b in range(_NB):
            j = g * _NB + b
            gather_wait(j, b)
            if not first:
                out_wait(j, b)
            transpose_chunk(b)
            if fire:
                gather_start(j + _NB, b)
            out_start(j, b)

    for b in range(_NB):
        gather_start(b, b)
    group(0, first=True, fire=True)
    lax.fori_loop(1, n_groups - 1,
                  lambda g, c: (group(g, first=False, fire=True), c)[1], 0)
    group(n_groups - 1, first=False, fire=False)
    for b in range(_NB):
        out_wait((n_groups - 1) * _NB + b, b)


def kernel(x, emb_weight):
    b0, b1 = x.shape
    total = b0 * b1
    n_chunks = b1
    assert b0 == _NW * _CHUNK and total == _NW * _CHUNK * n_chunks
    xi = x.reshape(_NW, _CHUNK, n_chunks).transpose(0, 2, 1).astype(jnp.int32)
    t3 = _prep(emb_weight.T).reshape(2 * _PAIR, _D)

    mesh = plsc.VectorSubcoreMesh(core_axis_name="c", subcore_axis_name="s")
    run = functools.partial(
        pl.kernel,
        out_type=jax.ShapeDtypeStruct((n_chunks * 8 * _NW, 1024), jnp.float32),
        mesh=mesh,
        scratch_types=[
            pltpu.VMEM((n_chunks, _CHUNK), jnp.int32),
            pltpu.VMEM((_NB, _CHUNK, _D), jnp.float32),
            pltpu.VMEM((_NB, 8, 1024), jnp.float32),
        ] + [pltpu.SemaphoreType.DMA] * (2 * _NB),
        compiler_params=pltpu.CompilerParams(use_tc_tiling_on_sc=False,
                                             needs_layout_passes=False),
    )(_emb_body)
    out2 = run(xi, t3)
    out5 = out2.reshape(n_chunks, 8, _NW, 8, _CHUNK)
    return out5.transpose(2, 4, 0, 1, 3).reshape(b0, b1, _D)


# speedup vs baseline: 2.3429x; 2.3429x over previous
import functools

import jax
import jax.numpy as jnp
from jax import lax
from jax.experimental import pallas as pl
from jax.experimental.pallas import tpu as pltpu
from jax.experimental.pallas import tpu_sc as plsc

_D = 64
_NW = 32
_CHUNK = 128
_NB = 4
_SCALE = 8.0
_PAIR = 500224


_PB = 512


def _prep_body(a_ref, b_ref, o_ref):
    o_ref[:, :_D] = a_ref[...].T * _SCALE
    o_ref[:, _D:] = b_ref[...].T * _SCALE


def _prep(wt):
    n_blocks = _PAIR // _PB
    return pl.pallas_call(
        _prep_body,
        grid=(n_blocks,),
        in_specs=[
            pl.BlockSpec((_D, _PB), lambda n: (0, n)),
            pl.BlockSpec((_D, _PB), lambda n: (0, n + _PAIR // _PB)),
        ],
        out_specs=pl.BlockSpec((_PB, 128), lambda n: (n, 0)),
        out_shape=jax.ShapeDtypeStruct((_PAIR, 128), jnp.float32),
    )(wt, wt)


def _emb_body(idx_hbm, table_hbm, out_hbm, idx_v, bufg, bufo, *sems):
    n_chunks = idx_v.shape[0]
    n_groups = n_chunks // _NB
    sem_g, sem_o = sems[:_NB], sems[_NB:]
    wid = lax.axis_index("s") * 2 + lax.axis_index("c")
    pltpu.sync_copy(idx_hbm.at[wid], idx_v)
    lane = lax.iota(jnp.int32, 16)

    def fix_idx(j, c):
        for g in range(_CHUNK // 16):
            s = pl.ds(g * 16, 16)
            v = idx_v[j, s]
            idx_v[j, s] = jnp.where(v < _PAIR, v + v, v + v - (2 * _PAIR - 1))
        return c

    lax.fori_loop(0, n_chunks, fix_idx, 0)

    def gather_start(j, b):
        pltpu.async_copy(table_hbm.at[idx_v.at[j]], bufg.at[b], sem_g[b])

    def gather_wait(j, b):
        pltpu.make_async_copy(table_hbm.at[idx_v.at[j]], bufg.at[b],
                              sem_g[b]).wait()

    def out_start(j, b):
        for a in range(8):
            pltpu.async_copy(bufo.at[b].at[a],
                             out_hbm.at[j * 256 + a * 32 + wid], sem_o[b])

    def out_wait(j, b):
        for a in range(8):
            pltpu.make_async_copy(bufo.at[b].at[a],
                                  out_hbm.at[j * 256 + a * 32 + wid],
                                  sem_o[b]).wait()

    def transpose_chunk(b):
        src, dst = bufg.at[b], bufo.at[b]

        def tile(q, c):
            c0 = (q & 7) * 16
            col0 = (q >> 3) * 16
            rows = c0 + lane
            for d in range(16):
                k = col0 + ((lane + d) & 15)
                vals = plsc.load_gather(src, [rows, k])
                plsc.store_scatter(dst, [k >> 3, ((k & 7) << 7) + rows], vals)
            return c

        lax.fori_loop(0, 32, tile, 0)

    def group(g, first, fire):
        for b in range(_NB):
            j = g * _NB + b
            gather_wait(j, b)
            if not first:
                out_wait(j, b)
            transpose_chunk(b)
            if fire:
                gather_start(j + _NB, b)
            out_start(j, b)

    for b in range(_NB):
        gather_start(b, b)
    group(0, first=True, fire=True)
    lax.fori_loop(1, n_groups - 1,
                  lambda g, c: (group(g, first=False, fire=True), c)[1], 0)
    group(n_groups - 1, first=False, fire=False)
    for b in range(_NB):
        out_wait((n_groups - 1) * _NB + b, b)


def kernel(x, emb_weight):
    b0, b1 = x.shape
    total = b0 * b1
    n_chunks = b1
    assert b0 == _NW * _CHUNK and total == _NW * _CHUNK * n_chunks
    xi = x.reshape(_NW, _CHUNK, n_chunks).transpose(0, 2, 1).astype(jnp.int32)
    t3 = _prep(emb_weight.T).reshape(2 * _PAIR, _D)

    mesh = plsc.VectorSubcoreMesh(core_axis_name="c", subcore_axis_name="s")
    run = functools.partial(
        pl.kernel,
        out_type=jax.ShapeDtypeStruct((n_chunks * 8 * _NW, 1024), jnp.float32),
        mesh=mesh,
        scratch_types=[
            pltpu.VMEM((n_chunks, _CHUNK), jnp.int32),
            pltpu.VMEM((_NB, _CHUNK, _D), jnp.float32),
            pltpu.VMEM((_NB, 8, 1024), jnp.float32),
        ] + [pltpu.SemaphoreType.DMA] * (2 * _NB),
        compiler_params=pltpu.CompilerParams(use_tc_tiling_on_sc=False,
                                             needs_layout_passes=False),
    )(_emb_body)
    out2 = run(xi, t3)
    out5 = out2.reshape(n_chunks, 8, _NW, 8, _CHUNK)
    return out5.transpose(2, 4, 0, 1, 3).reshape(b0, b1, _D)


# speedup vs baseline: 2.9951x; 1.2784x over previous
import functools

import jax
import jax.numpy as jnp
from jax import lax
from jax.experimental import pallas as pl
from jax.experimental.pallas import tpu as pltpu
from jax.experimental.pallas import tpu_sc as plsc

_D = 64
_NW = 32
_CHUNK = 128
_NB = 4
_SCALE = 8.0
_PAIR = 500736


_PB = 1024


def _prep_body(a_ref, b_ref, o_ref):
    o_ref[:, :_D] = a_ref[...].T * _SCALE
    o_ref[:, _D:] = b_ref[...].T * _SCALE


def _prep(wt):
    n_blocks = _PAIR // _PB
    return pl.pallas_call(
        _prep_body,
        grid=(n_blocks,),
        in_specs=[
            pl.BlockSpec((_D, _PB), lambda n: (0, n)),
            pl.BlockSpec(
                (_D, _PB),
                lambda n: (0, jnp.minimum(n + _PAIR // _PB,
                                          (1_000_000 - 1) // _PB)),
            ),
        ],
        out_specs=pl.BlockSpec((_PB, 128), lambda n: (n, 0)),
        out_shape=jax.ShapeDtypeStruct((_PAIR, 128), jnp.float32),
    )(wt, wt)


def _emb_body(idx_hbm, table_hbm, out_hbm, idx_v, bufg, bufo, *sems):
    n_chunks = idx_v.shape[0]
    n_groups = n_chunks // _NB
    sem_g, sem_o = sems[:_NB], sems[_NB:]
    wid = lax.axis_index("s") * 2 + lax.axis_index("c")
    pltpu.sync_copy(idx_hbm.at[wid], idx_v)
    lane = lax.iota(jnp.int32, 16)

    def fix_idx(j, c):
        for g in range(_CHUNK // 16):
            s = pl.ds(g * 16, 16)
            v = idx_v[j, s]
            idx_v[j, s] = jnp.where(v < _PAIR, v + v, v + v - (2 * _PAIR - 1))
        return c

    lax.fori_loop(0, n_chunks, fix_idx, 0)

    def gather_start(j, b):
        pltpu.async_copy(table_hbm.at[idx_v.at[j]], bufg.at[b], sem_g[b])

    def gather_wait(j, b):
        pltpu.make_async_copy(table_hbm.at[idx_v.at[j]], bufg.at[b],
                              sem_g[b]).wait()

    def out_start(j, b):
        for a in range(8):
            pltpu.async_copy(bufo.at[b].at[a],
                             out_hbm.at[j * 256 + a * 32 + wid], sem_o[b])

    def out_wait(j, b):
        for a in range(8):
            pltpu.make_async_copy(bufo.at[b].at[a],
                                  out_hbm.at[j * 256 + a * 32 + wid],
                                  sem_o[b]).wait()

    def transpose_chunk(b):
        src, dst = bufg.at[b], bufo.at[b]

        def tile(q, c):
            c0 = (q & 7) * 16
            col0 = (q >> 3) * 16
            rows = c0 + lane
            for d in range(16):
                k = col0 + ((lane + d) & 15)
                vals = plsc.load_gather(src, [rows, k])
                plsc.store_scatter(dst, [k >> 3, ((k & 7) << 7) + rows], vals)
            return c

        lax.fori_loop(0, 32, tile, 0)

    def group(g, first, fire):
        for b in range(_NB):
            j = g * _NB + b
            gather_wait(j, b)
            if not first:
                out_wait(j, b)
            transpose_chunk(b)
            if fire:
                gather_start(j + _NB, b)
            out_start(j, b)

    for b in range(_NB):
        gather_start(b, b)
    group(0, first=True, fire=True)
    lax.fori_loop(1, n_groups - 1,
                  lambda g, c: (group(g, first=False, fire=True), c)[1], 0)
    group(n_groups - 1, first=False, fire=False)
    for b in range(_NB):
        out_wait((n_groups - 1) * _NB + b, b)


def kernel(x, emb_weight):
    b0, b1 = x.shape
    total = b0 * b1
    n_chunks = b1
    assert b0 == _NW * _CHUNK and total == _NW * _CHUNK * n_chunks
    xi = x.reshape(_NW, _CHUNK, n_chunks).transpose(0, 2, 1).astype(jnp.int32)
    t3 = _prep(emb_weight.T).reshape(2 * _PAIR, _D)

    mesh = plsc.VectorSubcoreMesh(core_axis_name="c", subcore_axis_name="s")
    run = functools.partial(
        pl.kernel,
        out_type=jax.ShapeDtypeStruct((n_chunks * 8 * _NW, 1024), jnp.float32),
        mesh=mesh,
        scratch_types=[
            pltpu.VMEM((n_chunks, _CHUNK), jnp.int32),
            pltpu.VMEM((_NB, _CHUNK, _D), jnp.float32),
            pltpu.VMEM((_NB, 8, 1024), jnp.float32),
        ] + [pltpu.SemaphoreType.DMA] * (2 * _NB),
        compiler_params=pltpu.CompilerParams(use_tc_tiling_on_sc=False,
                                             needs_layout_passes=False),
    )(_emb_body)
    out2 = run(xi, t3)
    out5 = out2.reshape(n_chunks, 8, _NW, 8, _CHUNK)
    return out5.transpose(2, 4, 0, 1, 3).reshape(b0, b1, _D)


# speedup vs baseline: 3.5895x; 1.1985x over previous
import functools

import jax
import jax.numpy as jnp
from jax import lax
from jax.experimental import pallas as pl
from jax.experimental.pallas import tpu as pltpu
from jax.experimental.pallas import tpu_sc as plsc

_D = 64
_NW = 32
_CHUNK = 128
_NB = 4
_SCALE = 8.0
_PAIR = 501760


_PB = 2048


def _prep_body(a_ref, b_ref, o_ref):
    o_ref[:, :_D] = a_ref[...].T * _SCALE
    o_ref[:, _D:] = b_ref[...].T * _SCALE


def _prep(wt):
    n_blocks = _PAIR // _PB
    return pl.pallas_call(
        _prep_body,
        grid=(n_blocks,),
        in_specs=[
            pl.BlockSpec((_D, _PB), lambda n: (0, n)),
            pl.BlockSpec(
                (_D, _PB),
                lambda n: (0, jnp.minimum(n + _PAIR // _PB,
                                          (1_000_000 - 1) // _PB)),
            ),
        ],
        out_specs=pl.BlockSpec((_PB, 128), lambda n: (n, 0)),
        out_shape=jax.ShapeDtypeStruct((_PAIR, 128), jnp.float32),
    )(wt, wt)


def _emb_body(idx_hbm, table_hbm, out_hbm, idx_v, bufg, bufo, *sems):
    n_chunks = idx_v.shape[0]
    n_groups = n_chunks // _NB
    sem_g, sem_o = sems[:_NB], sems[_NB:]
    wid = lax.axis_index("s") * 2 + lax.axis_index("c")
    pltpu.sync_copy(idx_hbm.at[wid], idx_v)
    lane = lax.iota(jnp.int32, 16)

    def fix_idx(j, c):
        for g in range(_CHUNK // 16):
            s = pl.ds(g * 16, 16)
            v = idx_v[j, s]
            idx_v[j, s] = jnp.where(v < _PAIR, v + v, v + v - (2 * _PAIR - 1))
        return c

    lax.fori_loop(0, n_chunks, fix_idx, 0)

    def gather_start(j, b):
        pltpu.async_copy(table_hbm.at[idx_v.at[j]], bufg.at[b], sem_g[b])

    def gather_wait(j, b):
        pltpu.make_async_copy(table_hbm.at[idx_v.at[j]], bufg.at[b],
                              sem_g[b]).wait()

    def out_start(j, b):
        for a in range(8):
            pltpu.async_copy(bufo.at[b].at[a],
                             out_hbm.at[j * 256 + a * 32 + wid], sem_o[b])

    def out_wait(j, b):
        for a in range(8):
            pltpu.make_async_copy(bufo.at[b].at[a],
                                  out_hbm.at[j * 256 + a * 32 + wid],
                                  sem_o[b]).wait()

    def transpose_chunk(b):
        src, dst = bufg.at[b], bufo.at[b]

        def tile(q, c):
            c0 = (q & 7) * 16
            col0 = (q >> 3) * 16
            rows = c0 + lane
            for d in range(16):
                k = col0 + ((lane + d) & 15)
                vals = plsc.load_gather(src, [rows, k])
                plsc.store_scatter(dst, [k >> 3, ((k & 7) << 7) + rows], vals)
            return c

        lax.fori_loop(0, 32, tile, 0, unroll=2)

    def group(g, first, fire):
        for b in range(_NB):
            j = g * _NB + b
            gather_wait(j, b)
            if not first:
                out_wait(j, b)
            transpose_chunk(b)
            if fire:
                gather_start(j + _NB, b)
            out_start(j, b)

    for b in range(_NB):
        gather_start(b, b)
    group(0, first=True, fire=True)
    lax.fori_loop(1, n_groups - 1,
                  lambda g, c: (group(g, first=False, fire=True), c)[1], 0)
    group(n_groups - 1, first=False, fire=False)
    for b in range(_NB):
        out_wait((n_groups - 1) * _NB + b, b)


def kernel(x, emb_weight):
    b0, b1 = x.shape
    total = b0 * b1
    n_chunks = b1
    assert b0 == _NW * _CHUNK and total == _NW * _CHUNK * n_chunks
    xi = x.reshape(_NW, _CHUNK, n_chunks).transpose(0, 2, 1).astype(jnp.int32)
    t3 = _prep(emb_weight.T).reshape(2 * _PAIR, _D)

    mesh = plsc.VectorSubcoreMesh(core_axis_name="c", subcore_axis_name="s")
    run = functools.partial(
        pl.kernel,
        out_type=jax.ShapeDtypeStruct((n_chunks * 8 * _NW, 1024), jnp.float32),
        mesh=mesh,
        scratch_types=[
            pltpu.VMEM((n_chunks, _CHUNK), jnp.int32),
            pltpu.VMEM((_NB, _CHUNK, _D), jnp.float32),
            pltpu.VMEM((_NB, 8, 1024), jnp.float32),
        ] + [pltpu.SemaphoreType.DMA] * (2 * _NB),
        compiler_params=pltpu.CompilerParams(use_tc_tiling_on_sc=False,
                                             needs_layout_passes=False),
    )(_emb_body)
    out2 = run(xi, t3)
    out5 = out2.reshape(n_chunks, 8, _NW, 8, _CHUNK)
    return out5.transpose(2, 4, 0, 1, 3).reshape(b0, b1, _D)


# speedup vs baseline: 3.7952x; 1.0573x over previous
import functools

import jax
import jax.numpy as jnp
from jax import lax
from jax.experimental import pallas as pl
from jax.experimental.pallas import tpu as pltpu
from jax.experimental.pallas import tpu_sc as plsc

_D = 64
_NW = 32
_CHUNK = 128
_NB = 4
_SCALE = 8.0
_PAIR = 503808


_PB = 4096


def _prep_body(a_ref, b_ref, o_ref):
    o_ref[:, :_D] = a_ref[...].T * _SCALE
    o_ref[:, _D:] = b_ref[...].T * _SCALE


def _prep(wt):
    n_blocks = _PAIR // _PB
    return pl.pallas_call(
        _prep_body,
        grid=(n_blocks,),
        in_specs=[
            pl.BlockSpec((_D, _PB), lambda n: (0, n)),
            pl.BlockSpec(
                (_D, _PB),
                lambda n: (0, jnp.minimum(n + _PAIR // _PB,
                                          (1_000_000 - 1) // _PB)),
            ),
        ],
        out_specs=pl.BlockSpec((_PB, 128), lambda n: (n, 0)),
        out_shape=jax.ShapeDtypeStruct((_PAIR, 128), jnp.float32),
    )(wt, wt)


def _emb_body(idx_hbm, table_hbm, out_hbm, idx_v, bufg, bufo, *sems):
    n_chunks = idx_v.shape[0]
    n_groups = n_chunks // _NB
    sem_g, sem_o = sems[:_NB], sems[_NB:]
    wid = lax.axis_index("s") * 2 + lax.axis_index("c")
    pltpu.sync_copy(idx_hbm.at[wid], idx_v)
    lane = lax.iota(jnp.int32, 16)

    def fix_idx(j, c):
        for g in range(_CHUNK // 16):
            s = pl.ds(g * 16, 16)
            v = idx_v[j, s]
            idx_v[j, s] = jnp.where(v < _PAIR, v + v, v + v - (2 * _PAIR - 1))
        return c

    lax.fori_loop(0, n_chunks, fix_idx, 0)

    def gather_start(j, b):
        pltpu.async_copy(table_hbm.at[idx_v.at[j]], bufg.at[b], sem_g[b])

    def gather_wait(j, b):
        pltpu.make_async_copy(table_hbm.at[idx_v.at[j]], bufg.at[b],
                              sem_g[b]).wait()

    def out_start(j, b):
        for a in range(8):
            pltpu.async_copy(bufo.at[b].at[a],
                             out_hbm.at[j * 256 + a * 32 + wid], sem_o[b])

    def out_wait(j, b):
        for a in range(8):
            pltpu.make_async_copy(bufo.at[b].at[a],
                                  out_hbm.at[j * 256 + a * 32 + wid],
                                  sem_o[b]).wait()

    def transpose_chunk(b):
        src, dst = bufg.at[b], bufo.at[b]

        def tile(q, c):
            c0 = (q & 7) * 16
            col0 = (q >> 3) * 16
            rows = c0 + lane
            for d in range(16):
                k = col0 + ((lane + d) & 15)
                vals = plsc.load_gather(src, [rows, k])
                plsc.store_scatter(dst, [k >> 3, ((k & 7) << 7) + rows], vals)
            return c

        lax.fori_loop(0, 32, tile, 0, unroll=4)

    def group(g, first, fire):
        for b in range(_NB):
            j = g * _NB + b
            gather_wait(j, b)
            if not first:
                out_wait(j, b)
            transpose_chunk(b)
            if fire:
                gather_start(j + _NB, b)
            out_start(j, b)

    for b in range(_NB):
        gather_start(b, b)
    group(0, first=True, fire=True)
    lax.fori_loop(1, n_groups - 1,
                  lambda g, c: (group(g, first=False, fire=True), c)[1], 0)
    group(n_groups - 1, first=False, fire=False)
    for b in range(_NB):
        out_wait((n_groups - 1) * _NB + b, b)


def kernel(x, emb_weight):
    b0, b1 = x.shape
    total = b0 * b1
    n_chunks = b1
    assert b0 == _NW * _CHUNK and total == _NW * _CHUNK * n_chunks
    xi = x.reshape(_NW, _CHUNK, n_chunks).transpose(0, 2, 1).astype(jnp.int32)
    t3 = _prep(emb_weight.T).reshape(2 * _PAIR, _D)

    mesh = plsc.VectorSubcoreMesh(core_axis_name="c", subcore_axis_name="s")
    run = functools.partial(
        pl.kernel,
        out_type=jax.ShapeDtypeStruct((n_chunks * 8 * _NW, 1024), jnp.float32),
        mesh=mesh,
        scratch_types=[
            pltpu.VMEM((n_chunks, _CHUNK), jnp.int32),
            pltpu.VMEM((_NB, _CHUNK, _D), jnp.float32),
            pltpu.VMEM((_NB, 8, 1024), jnp.float32),
        ] + [pltpu.SemaphoreType.DMA] * (2 * _NB),
        compiler_params=pltpu.CompilerParams(use_tc_tiling_on_sc=False,
                                             needs_layout_passes=False),
    )(_emb_body)
    out2 = run(xi, t3)
    out5 = out2.reshape(n_chunks, 8, _NW, 8, _CHUNK)
    return out5.transpose(2, 4, 0, 1, 3).reshape(b0, b1, _D)


# speedup vs baseline: 3.8284x; 1.0087x over previous
import functools

import jax
import jax.numpy as jnp
from jax import lax
from jax.experimental import pallas as pl
from jax.experimental.pallas import tpu as pltpu
from jax.experimental.pallas import tpu_sc as plsc

_D = 64
_NW = 32
_CHUNK = 128
_NB = 4
_SCALE = 8.0
_PAIR = 503808


_PB = 4096


def _prep_body(a_ref, b_ref, o_ref):
    o_ref[:, :_D] = a_ref[...].T * _SCALE
    o_ref[:, _D:] = b_ref[...].T * _SCALE


def _prep(wt):
    n_blocks = _PAIR // _PB
    return pl.pallas_call(
        _prep_body,
        grid=(n_blocks,),
        in_specs=[
            pl.BlockSpec((_D, _PB), lambda n: (0, n)),
            pl.BlockSpec(
                (_D, _PB),
                lambda n: (0, jnp.minimum(n + _PAIR // _PB,
                                          (1_000_000 - 1) // _PB)),
            ),
        ],
        out_specs=pl.BlockSpec((_PB, 128), lambda n: (n, 0)),
        out_shape=jax.ShapeDtypeStruct((_PAIR, 128), jnp.float32),
    )(wt, wt)


def _emb_body(idx_hbm, table_hbm, out_hbm, idx_v, bufg, bufo, *sems):
    n_chunks = idx_v.shape[0]
    n_groups = n_chunks // _NB
    sem_g, sem_o = sems[:_NB], sems[_NB:]
    wid = lax.axis_index("s") * 2 + lax.axis_index("c")
    pltpu.sync_copy(idx_hbm.at[wid], idx_v)
    lane = lax.iota(jnp.int32, 16)

    def fix_idx(j, c):
        for g in range(_CHUNK // 16):
            s = pl.ds(g * 16, 16)
            v = idx_v[j, s]
            idx_v[j, s] = jnp.where(v < _PAIR, v + v, v + v - (2 * _PAIR - 1))
        return c

    lax.fori_loop(0, n_chunks, fix_idx, 0)

    def gather_start(j, b):
        pltpu.async_copy(table_hbm.at[idx_v.at[j]], bufg.at[b], sem_g[b])

    def gather_wait(j, b):
        pltpu.make_async_copy(table_hbm.at[idx_v.at[j]], bufg.at[b],
                              sem_g[b]).wait()

    def out_start(j, b):
        for a in range(8):
            pltpu.async_copy(bufo.at[b].at[a],
                             out_hbm.at[j * 256 + a * 32 + wid], sem_o[b])

    def out_wait(j, b):
        pltpu.make_async_copy(out_hbm.at[pl.ds(j * 256, 8)], bufo.at[b],
                              sem_o[b]).wait()

    def transpose_chunk(b):
        src, dst = bufg.at[b], bufo.at[b]

        def tile(q, c):
            c0 = (q & 7) * 16
            col0 = (q >> 3) * 16
            rows = c0 + lane
            for d in range(16):
                k = col0 + ((lane + d) & 15)
                vals = plsc.load_gather(src, [rows, k])
                plsc.store_scatter(dst, [k >> 3, ((k & 7) << 7) + rows], vals)
            return c

        lax.fori_loop(0, 32, tile, 0, unroll=4)

    def group(g, first, fire):
        for b in range(_NB):
            j = g * _NB + b
            gather_wait(j, b)
            if not first:
                out_wait(j, b)
            transpose_chunk(b)
            if fire:
                gather_start(j + _NB, b)
            out_start(j, b)

    for b in range(_NB):
        gather_start(b, b)
    group(0, first=True, fire=True)
    lax.fori_loop(1, n_groups - 1,
                  lambda g, c: (group(g, first=False, fire=True), c)[1], 0)
    group(n_groups - 1, first=False, fire=False)
    for b in range(_NB):
        out_wait((n_groups - 1) * _NB + b, b)


def kernel(x, emb_weight):
    b0, b1 = x.shape
    total = b0 * b1
    n_chunks = b1
    assert b0 == _NW * _CHUNK and total == _NW * _CHUNK * n_chunks
    xi = x.reshape(_NW, _CHUNK, n_chunks).transpose(0, 2, 1).astype(jnp.int32)
    t3 = _prep(emb_weight.T).reshape(2 * _PAIR, _D)

    mesh = plsc.VectorSubcoreMesh(core_axis_name="c", subcore_axis_name="s")
    run = functools.partial(
        pl.kernel,
        out_type=jax.ShapeDtypeStruct((n_chunks * 8 * _NW, 1024), jnp.float32),
        mesh=mesh,
        scratch_types=[
            pltpu.VMEM((n_chunks, _CHUNK), jnp.int32),
            pltpu.VMEM((_NB, _CHUNK, _D), jnp.float32),
            pltpu.VMEM((_NB, 8, 1024), jnp.float32),
        ] + [pltpu.SemaphoreType.DMA] * (2 * _NB),
        compiler_params=pltpu.CompilerParams(use_tc_tiling_on_sc=False,
                                             needs_layout_passes=False),
    )(_emb_body)
    out2 = run(xi, t3)
    out5 = out2.reshape(n_chunks, 8, _NW, 8, _CHUNK)
    return out5.transpose(2, 4, 0, 1, 3).reshape(b0, b1, _D)
